# baseline (speedup 1.0000x reference)
import functools

import jax
import jax.numpy as jnp
import numpy as np
from jax import lax
from jax.experimental import pallas as pl
from jax.experimental.pallas import tpu as pltpu
from jax.experimental.pallas import tpu_sc as plsc

_B = 4096
_ND = 13
_NS = 26
_EMB = 16
_VOCAB = 1000000
_FIELD = 38461
_REG = 1e-4
_TCHUNK = 8192


def _relayout_table(table):
  tt = table.T
  eye = jnp.eye(_EMB, dtype=jnp.float32)

  def body(tt_ref, eye_ref, out_ref):
    out_ref[...] = lax.dot_general(
        tt_ref[...], eye_ref[...], (((0,), (0,)), ((), ())),
        preferred_element_type=jnp.float32)

  return pl.pallas_call(
      body,
      grid=(pl.cdiv(_VOCAB, _TCHUNK),),
      in_specs=[
          pl.BlockSpec((_EMB, _TCHUNK), lambda i: (0, i)),
          pl.BlockSpec((_EMB, _EMB), lambda i: (0, 0)),
      ],
      out_specs=pl.BlockSpec((_TCHUNK, _EMB), lambda i: (i, 0)),
      out_shape=jax.ShapeDtypeStruct((_VOCAB, _EMB), jnp.float32),
  )(tt, eye)


def _make_sc_kernel(nc, ns, per):
  nw = nc * ns
  mesh = plsc.VectorSubcoreMesh(core_axis_name="c", subcore_axis_name="s")

  @functools.partial(
      pl.kernel,
      mesh=mesh,
      compiler_params=pltpu.CompilerParams(
          use_tc_tiling_on_sc=False, needs_layout_passes=False),
      out_type=(
          jax.ShapeDtypeStruct((_B, _EMB), jnp.float32),
          jax.ShapeDtypeStruct((nw, 2, _EMB), jnp.float32),
      ),
      scratch_types=[
          pltpu.VMEM((per, _NS), jnp.int32),
          pltpu.VMEM((per, _ND), jnp.float32),
          pltpu.VMEM((_NS, per), jnp.int32),
          pltpu.VMEM((_NS, per, _EMB), jnp.float32),
          pltpu.VMEM((_NS + 2, _EMB), jnp.float32),
          pltpu.VMEM((per, _EMB), jnp.float32),
          pltpu.VMEM((2, _EMB), jnp.float32),
          pltpu.SemaphoreType.DMA,
      ],
  )
  def sc_kernel(rs_hbm, rd_hbm, table_hbm, wconst_hbm,
                out_vec, out_sq,
                rs_v, rd_v, idx_v, rows_v, wc_v, acc_v, sq_v,
                sem):
    wid = lax.axis_index("s") * nc + lax.axis_index("c")
    base = wid * per

    pltpu.sync_copy(rs_hbm.at[pl.ds(base, per)], rs_v)
    pltpu.sync_copy(rd_hbm.at[pl.ds(base, per)], rd_v)
    pltpu.sync_copy(wconst_hbm, wc_v)

    lane = lax.iota(jnp.int32, _EMB)
    for s in range(_NS):
      col = jnp.full((_EMB,), s, jnp.int32)
      off = _ND + s * _FIELD
      for g in range(per // _EMB):
        rows = lane + g * _EMB
        idx_v[s, pl.ds(g * _EMB, _EMB)] = (
            plsc.load_gather(rs_v, [rows, col]) + off)

    copies = [
        pltpu.make_async_copy(table_hbm.at[idx_v.at[s]], rows_v.at[s], sem)
        for s in range(_NS)
    ]
    for cp in copies:
      cp.start()
    for cp in copies:
      cp.wait()

    ws = [wc_v[s] for s in range(_NS)]
    cvec = wc_v[_NS]
    nvec = wc_v[_NS + 1]
    colsafe = jnp.minimum(lane, _ND - 1)
    zero = jnp.zeros((_EMB,), jnp.float32)

    def body(bi, carry):
      dsq, ssq = carry
      rd = plsc.load_gather(rd_v, [jnp.full((_EMB,), bi, jnp.int32), colsafe])
      acc = rd * cvec
      dsq = dsq + rd * rd * nvec
      for s in range(_NS):
        row = rows_v[s, bi]
        acc = acc + row * ws[s]
        ssq = ssq + row * row
      acc_v[bi] = acc
      return dsq, ssq

    dsq, ssq = lax.fori_loop(0, per, body, (zero, zero))
    sq_v[0] = dsq
    sq_v[1] = ssq

    pltpu.sync_copy(acc_v, out_vec.at[pl.ds(base, per)])
    pltpu.sync_copy(sq_v, out_sq.at[wid])

  return sc_kernel


def kernel(raw_dense, raw_sparse, table, W, b):
  info = plsc.get_sparse_core_info()
  nc, ns = info.num_cores, info.num_subcores
  nw = nc * ns
  per = _B // nw

  wf = W[:, 0]
  ws = wf[_ND * _EMB:].reshape(_NS, _EMB)
  wd = wf[:_ND * _EMB].reshape(_ND, _EMB)
  cvec = jnp.pad((table[:_ND] * wd).sum(axis=1), (0, _EMB - _ND))
  nvec = jnp.pad((table[:_ND] ** 2).sum(axis=1), (0, _EMB - _ND))
  wconst = jnp.concatenate([ws, cvec[None, :], nvec[None, :]], axis=0)

  t2 = _relayout_table(table)
  out_vec, sq = _make_sc_kernel(nc, ns, per)(raw_sparse, raw_dense, t2,
                                             wconst)

  logits = out_vec.sum(axis=1, keepdims=True) + b[0]
  norms = jnp.sqrt(sq[:, 0, :].sum()) + jnp.sqrt(sq[:, 1, :].sum())
  regs = _REG * norms
  return logits, regs


# speedup vs baseline: 2.2804x; 2.2804x over previous
import functools

import jax
import jax.numpy as jnp
import numpy as np
from jax import lax
from jax.experimental import pallas as pl
from jax.experimental.pallas import tpu as pltpu
from jax.experimental.pallas import tpu_sc as plsc

_B = 4096
_ND = 13
_NS = 26
_EMB = 16
_VOCAB = 1000000
_FIELD = 38461
_REG = 1e-4
_TCHUNK = 8192
_NBLK = (_VOCAB + _TCHUNK - 1) // _TCHUNK
_VPAD = _NBLK * _TCHUNK
_PROWS = 32


def _field_dots(table, wse):
  tt = table.T

  def body(tt_ref, w_ref, out_ref):
    x = tt_ref[...]
    out_ref[...] = lax.dot_general(
        w_ref[...], x, (((1,), (0,)), ((), ())),
        preferred_element_type=jnp.float32)
    out_ref[_NS:_NS + 1, :] = jnp.sum(x * x, axis=0, keepdims=True)

  return pl.pallas_call(
      body,
      grid=(_NBLK,),
      in_specs=[
          pl.BlockSpec((_EMB, _TCHUNK), lambda i: (0, i)),
          pl.BlockSpec((_PROWS, _EMB), lambda i: (0, 0)),
      ],
      out_specs=pl.BlockSpec((_PROWS, _TCHUNK), lambda i: (0, i)),
      out_shape=jax.ShapeDtypeStruct((_PROWS, _VPAD), jnp.float32),
  )(tt, wse)


def _make_sc_kernel(nc, ns, per):
  nw = nc * ns
  ng = _NS * 2
  mesh = plsc.VectorSubcoreMesh(core_axis_name="c", subcore_axis_name="s")

  @functools.partial(
      pl.kernel,
      mesh=mesh,
      compiler_params=pltpu.CompilerParams(
          use_tc_tiling_on_sc=False, needs_layout_passes=False),
      out_type=(
          jax.ShapeDtypeStruct((_B,), jnp.float32),
          jax.ShapeDtypeStruct((nw, 2, _EMB), jnp.float32),
      ),
      scratch_types=[
          pltpu.VMEM((per, _NS), jnp.int32),
          pltpu.VMEM((per, _ND), jnp.float32),
          pltpu.VMEM((ng, per), jnp.int32),
          pltpu.VMEM((ng, per), jnp.float32),
          pltpu.VMEM((2, _EMB), jnp.float32),
          pltpu.VMEM((per,), jnp.float32),
          pltpu.VMEM((2, _EMB), jnp.float32),
          pltpu.SemaphoreType.DMA,
      ],
  )
  def sc_kernel(rs_hbm, rd_hbm, pflat_hbm, wconst_hbm,
                out_logits, out_sq,
                rs_v, rd_v, idx_v, g_v, wc_v, logits_v, sq_v,
                sem):
    wid = lax.axis_index("s") * nc + lax.axis_index("c")
    base = wid * per

    pltpu.sync_copy(rs_hbm.at[pl.ds(base, per)], rs_v)
    pltpu.sync_copy(rd_hbm.at[pl.ds(base, per)], rd_v)
    pltpu.sync_copy(wconst_hbm, wc_v)

    lane = lax.iota(jnp.int32, _EMB)
    for s in range(_NS):
      col = jnp.full((_EMB,), s, jnp.int32)
      off = _ND + s * _FIELD
      for g in range(per // _EMB):
        rows = lane + g * _EMB
        v = plsc.load_gather(rs_v, [rows, col]) + off
        sl = pl.ds(g * _EMB, _EMB)
        idx_v[s, sl] = v + s * _VPAD
        idx_v[_NS + s, sl] = v + _NS * _VPAD

    copies = [
        pltpu.make_async_copy(pflat_hbm.at[idx_v.at[k]], g_v.at[k], sem)
        for k in range(ng)
    ]
    for cp in copies:
      cp.start()
    for cp in copies:
      cp.wait()

    zero = jnp.zeros((_EMB,), jnp.float32)
    cv = wc_v[0]
    nv = wc_v[1]
    dsq = zero
    ssq = zero
    for g in range(per // _EMB):
      sl = pl.ds(g * _EMB, _EMB)
      rows = lane + g * _EMB
      acc = zero
      for s in range(_NS):
        acc = acc + g_v[s, sl]
        ssq = ssq + g_v[_NS + s, sl]
      for d in range(_ND):
        cold = jnp.full((_EMB,), d, jnp.int32)
        rd16 = plsc.load_gather(rd_v, [rows, cold])
        acc = acc + rd16 * cv[d]
        dsq = dsq + rd16 * rd16 * nv[d]
      logits_v[sl] = acc
    sq_v[0] = dsq
    sq_v[1] = ssq

    pltpu.sync_copy(logits_v, out_logits.at[pl.ds(base, per)])
    pltpu.sync_copy(sq_v, out_sq.at[wid])

  return sc_kernel


def kernel(raw_dense, raw_sparse, table, W, b):
  info = plsc.get_sparse_core_info()
  nc, ns = info.num_cores, info.num_subcores
  nw = nc * ns
  per = _B // nw

  wf = W[:, 0]
  ws = wf[_ND * _EMB:].reshape(_NS, _EMB)
  wse = jnp.zeros((_PROWS, _EMB), jnp.float32).at[:_NS].set(ws)
  wd = wf[:_ND * _EMB].reshape(_ND, _EMB)
  cvec = jnp.pad((table[:_ND] * wd).sum(axis=1), (0, _EMB - _ND))
  nvec = jnp.pad((table[:_ND] ** 2).sum(axis=1), (0, _EMB - _ND))
  wconst = jnp.stack([cvec, nvec])

  p = _field_dots(table, wse)
  pflat = jnp.reshape(p, (-1,))

  logits_flat, sq = _make_sc_kernel(nc, ns, per)(
      raw_sparse, raw_dense, pflat, wconst)

  logits = logits_flat[:, None] + b[0]
  norms = jnp.sqrt(sq[:, 0, :].sum()) + jnp.sqrt(sq[:, 1, :].sum())
  regs = _REG * norms
  return logits, regs


# speedup vs baseline: 3.5247x; 1.5457x over previous
import functools

import jax
import jax.numpy as jnp
import numpy as np
from jax import lax
from jax.experimental import pallas as pl
from jax.experimental.pallas import tpu as pltpu
from jax.experimental.pallas import tpu_sc as plsc

_B = 4096
_ND = 13
_NS = 26
_EMB = 16
_VOCAB = 1000000
_FIELD = 38461
_REG = 1e-4
_TCHUNK = 8192
_NBLK = (_VOCAB + _TCHUNK - 1) // _TCHUNK
_VPAD = _NBLK * _TCHUNK
_PROWS = 32


def _field_dots(table, wse):
  tt = table.T
  nvt = _TCHUNK // 128

  def body(tt_ref, w_ref, out_ref):
    x = tt_ref[...]
    for sg in range(4):
      m = lax.dot_general(
          w_ref[8 * sg:8 * (sg + 1), :], x, (((1,), (0,)), ((), ())),
          preferred_element_type=jnp.float32)
      if sg == _NS // 8:
        sub = lax.broadcasted_iota(jnp.int32, (8, _TCHUNK), 0)
        nrow = jnp.broadcast_to(jnp.sum(x * x, axis=0, keepdims=True),
                                (8, _TCHUNK))
        m = jnp.where(sub == _NS % 8, nrow, m)
      for j in range(nvt):
        out_ref[sg, j] = m[:, 128 * j:128 * (j + 1)]

  return pl.pallas_call(
      body,
      grid=(_NBLK,),
      in_specs=[
          pl.BlockSpec((_EMB, _TCHUNK), lambda i: (0, i)),
          pl.BlockSpec((_PROWS, _EMB), lambda i: (0, 0)),
      ],
      out_specs=pl.BlockSpec((4, nvt, 8, 128), lambda i: (0, i, 0, 0)),
      out_shape=jax.ShapeDtypeStruct((4, _VPAD // 128, 8, 128), jnp.float32),
  )(tt, wse)


def _make_sc_kernel(nc, ns, per):
  nw = nc * ns
  ng = _NS * 2
  mesh = plsc.VectorSubcoreMesh(core_axis_name="c", subcore_axis_name="s")

  @functools.partial(
      pl.kernel,
      mesh=mesh,
      compiler_params=pltpu.CompilerParams(
          use_tc_tiling_on_sc=False, needs_layout_passes=False),
      out_type=(
          jax.ShapeDtypeStruct((_B,), jnp.float32),
          jax.ShapeDtypeStruct((nw, 2, _EMB), jnp.float32),
      ),
      scratch_types=[
          pltpu.VMEM((per, _NS), jnp.int32),
          pltpu.VMEM((per, _ND), jnp.float32),
          pltpu.VMEM((ng, per), jnp.int32),
          pltpu.VMEM((ng, per), jnp.float32),
          pltpu.VMEM((2, _EMB), jnp.float32),
          pltpu.VMEM((per,), jnp.float32),
          pltpu.VMEM((2, _EMB), jnp.float32),
          pltpu.SemaphoreType.DMA,
      ],
  )
  def sc_kernel(rs_hbm, rd_hbm, pflat_hbm, wconst_hbm,
                out_logits, out_sq,
                rs_v, rd_v, idx_v, g_v, wc_v, logits_v, sq_v,
                sem):
    wid = lax.axis_index("s") * nc + lax.axis_index("c")
    base = wid * per

    pltpu.sync_copy(rs_hbm.at[pl.ds(base, per)], rs_v)
    pltpu.sync_copy(rd_hbm.at[pl.ds(base, per)], rd_v)
    pltpu.sync_copy(wconst_hbm, wc_v)

    lane = lax.iota(jnp.int32, _EMB)
    an = (_NS >> 3) * 8 * _VPAD + (_NS & 7) * 128
    for s in range(_NS):
      col = jnp.full((_EMB,), s, jnp.int32)
      off = _ND + s * _FIELD
      a = (s >> 3) * 8 * _VPAD + (s & 7) * 128
      for g in range(per // _EMB):
        rows = lane + g * _EMB
        v = plsc.load_gather(rs_v, [rows, col]) + off
        t = ((v >> 7) << 10) | (v & 127)
        sl = pl.ds(g * _EMB, _EMB)
        idx_v[s, sl] = t + a
        idx_v[_NS + s, sl] = t + an

    copies = [
        pltpu.make_async_copy(pflat_hbm.at[idx_v.at[k]], g_v.at[k], sem)
        for k in range(ng)
    ]
    for cp in copies:
      cp.start()
    for cp in copies:
      cp.wait()

    zero = jnp.zeros((_EMB,), jnp.float32)
    cv = wc_v[0]
    nv = wc_v[1]
    dsq = zero
    ssq = zero
    for g in range(per // _EMB):
      sl = pl.ds(g * _EMB, _EMB)
      rows = lane + g * _EMB
      acc = zero
      for s in range(_NS):
        acc = acc + g_v[s, sl]
        ssq = ssq + g_v[_NS + s, sl]
      for d in range(_ND):
        cold = jnp.full((_EMB,), d, jnp.int32)
        rd16 = plsc.load_gather(rd_v, [rows, cold])
        acc = acc + rd16 * cv[d]
        dsq = dsq + rd16 * rd16 * nv[d]
      logits_v[sl] = acc
    sq_v[0] = dsq
    sq_v[1] = ssq

    pltpu.sync_copy(logits_v, out_logits.at[pl.ds(base, per)])
    pltpu.sync_copy(sq_v, out_sq.at[wid])

  return sc_kernel


def kernel(raw_dense, raw_sparse, table, W, b):
  info = plsc.get_sparse_core_info()
  nc, ns = info.num_cores, info.num_subcores
  nw = nc * ns
  per = _B // nw

  wf = W[:, 0]
  ws = wf[_ND * _EMB:].reshape(_NS, _EMB)
  wse = jnp.zeros((_PROWS, _EMB), jnp.float32).at[:_NS].set(ws)
  wd = wf[:_ND * _EMB].reshape(_ND, _EMB)
  cvec = jnp.pad((table[:_ND] * wd).sum(axis=1), (0, _EMB - _ND))
  nvec = jnp.pad((table[:_ND] ** 2).sum(axis=1), (0, _EMB - _ND))
  wconst = jnp.stack([cvec, nvec])

  p = _field_dots(table, wse)
  pflat = jnp.reshape(p, (-1,))

  logits_flat, sq = _make_sc_kernel(nc, ns, per)(
      raw_sparse, raw_dense, pflat, wconst)

  logits = logits_flat[:, None] + b[0]
  norms = jnp.sqrt(sq[:, 0, :].sum()) + jnp.sqrt(sq[:, 1, :].sum())
  regs = _REG * norms
  return logits, regs
